# baseline (speedup 1.0000x reference)
import functools

import jax
import jax.numpy as jnp
from jax import lax
from jax.experimental import pallas as pl
from jax.experimental.pallas import tpu as pltpu
from jax.experimental.pallas import tpu_sc as plsc

N = 10000
E = 320000
D = 128
NC, NS = 2, 16
NW = NC * NS
EW = E // NW
K = 80
CH = EW // K
RPT = N // NS


def _sc_mesh():
    return plsc.VectorSubcoreMesh(
        core_axis_name="c", subcore_axis_name="s", num_cores=NC, num_subcores=NS
    )


OUT_OFF = 624
OUT_WIN = 640
XCH = 125
XNJ = RPT // XCH


@functools.cache
def _get_deg_kernel():
    return functools.partial(
        pl.kernel,
        out_type=(
            jax.ShapeDtypeStruct((NC * N,), jnp.float32),
            jax.ShapeDtypeStruct((NS, XNJ, XCH, D), jnp.float32),
        ),
        mesh=_sc_mesh(),
        scratch_types=[
            pltpu.VMEM((CH, K), jnp.int32),
            pltpu.VMEM((K,), jnp.float32),
            pltpu.VMEM((XNJ, XCH), jnp.int32),
            pltpu.VMEM((XCH, D), jnp.float32),
            pltpu.VMEM((OUT_WIN,), jnp.float32),
            pltpu.VMEM_SHARED((N,), jnp.float32),
            pltpu.SemaphoreType.DMA,
        ],
    )(_deg_body)


def _deg_body(dst_hbm, x_hbm, emb_hbm, ones_hbm, deg_out, emb_out,
              dst_v, ones_v, x_v, erows_v, stage_v, acc_sh, sem):
    c = lax.axis_index("c")
    s = lax.axis_index("s")
    wid = c * NS + s
    pltpu.sync_copy(dst_hbm.at[wid], dst_v)
    pltpu.sync_copy(ones_hbm, ones_v)
    for j in range(OUT_WIN // 16):
        stage_v[pl.ds(16 * j, 16)] = jnp.zeros((16,), jnp.float32)
    pltpu.sync_copy(stage_v, acc_sh.at[pl.ds(s * OUT_OFF, OUT_WIN)])

    @pl.when(c == 0)
    def _():
        pltpu.sync_copy(x_hbm.at[s], x_v)

    plsc.subcore_barrier()

    @pl.loop(0, CH)
    def _(i):
        pltpu.sync_copy(ones_v, acc_sh.at[dst_v.at[i]], add=True)

    @pl.when(c == 0)
    def _():
        for j in range(XNJ):
            pltpu.async_copy(emb_hbm.at[x_v.at[j]], erows_v, sem).wait()
            pltpu.sync_copy(erows_v, emb_out.at[s, j])

    plsc.subcore_barrier()
    pltpu.sync_copy(acc_sh.at[pl.ds(s * OUT_OFF, OUT_WIN)], stage_v)
    pltpu.sync_copy(stage_v, deg_out.at[pl.ds(c * N + s * OUT_OFF, OUT_WIN)])


@functools.cache
def _get_msgpass_kernel():
    return functools.partial(
        pl.kernel,
        out_type=jax.ShapeDtypeStruct((NC, NS, RPT, D), jnp.float32),
        mesh=_sc_mesh(),
        scratch_types=[
            pltpu.VMEM((EW,), jnp.int32),
            pltpu.VMEM((CH, K), jnp.int32),
            pltpu.VMEM((K, D), jnp.float32),
            pltpu.VMEM((K, D), jnp.float32),
            pltpu.VMEM_SHARED((N, D), jnp.float32),
            pltpu.SemaphoreType.DMA,
            pltpu.SemaphoreType.DMA,
            pltpu.SemaphoreType.DMA,
            pltpu.SemaphoreType.DMA,
        ],
    )(_msgpass_body)


def _msgpass_body(h_hbm, h_blk_hbm, src_hbm, dst_hbm, zero_hbm, out_hbm,
                  src_v, dst_v, rows_a, rows_b, acc_sh, sem_a, sem_b,
                  sem_sa, sem_sb):
    c = lax.axis_index("c")
    s = lax.axis_index("s")
    wid = c * NS + s
    pltpu.sync_copy(src_hbm.at[pl.ds(wid * EW, EW)], src_v)
    pltpu.sync_copy(dst_hbm.at[wid], dst_v)
    @pl.when(c == 0)
    def _():
        pltpu.sync_copy(h_blk_hbm.at[s], acc_sh.at[pl.ds(s * RPT, RPT)])

    @pl.when(c != 0)
    def _():
        pltpu.sync_copy(zero_hbm.at[s], acc_sh.at[pl.ds(s * RPT, RPT)])

    plsc.subcore_barrier()

    def _gather(i, buf, sem):
        pltpu.async_copy(h_hbm.at[src_v.at[pl.ds(i * K, K)]], buf, sem)

    def _gwait(i, buf, sem):
        pltpu.make_async_copy(
            h_hbm.at[src_v.at[pl.ds(i * K, K)]], buf, sem).wait()

    def _scat(i, buf, sem):
        pltpu.async_copy(buf, acc_sh.at[dst_v.at[i]], sem, add=True)

    def _swait(i, buf, sem):
        pltpu.make_async_copy(buf, acc_sh.at[dst_v.at[i]], sem).wait()

    _gather(0, rows_a, sem_a)
    _gather(1, rows_b, sem_b)

    @pl.loop(0, CH // 2)
    def _(t):
        i0 = 2 * t
        _gwait(i0, rows_a, sem_a)
        _scat(i0, rows_a, sem_sa)
        _gwait(i0 + 1, rows_b, sem_b)
        _scat(i0 + 1, rows_b, sem_sb)
        _swait(i0, rows_a, sem_sa)
        _gather(i0 + 2, rows_a, sem_a)
        _swait(i0 + 1, rows_b, sem_sb)

        @pl.when(t < CH // 2 - 1)
        def _():
            _gather(i0 + 3, rows_b, sem_b)

    _gwait(CH - 1, rows_a, sem_a)
    pltpu.sync_copy(rows_a, acc_sh.at[dst_v.at[CH - 1]], add=True)

    plsc.subcore_barrier()
    pltpu.sync_copy(acc_sh.at[pl.ds(s * RPT, RPT)], out_hbm.at[c, s])


def _dinv_block(d0_ref, d1_ref):
    d = 1.0 + d0_ref[0] + d1_ref[0]
    return lax.rsqrt(d)


def _mm_scale_body(emb_ref, w_ref, d0_ref, d1_ref, out_ref):
    dinv = _dinv_block(d0_ref, d1_ref)
    h = jnp.dot(emb_ref[...], w_ref[...], preferred_element_type=jnp.float32)
    out_ref[...] = h * dinv


def _mid_body(s_ref, d0_ref, d1_ref, b_ref, w_ref, out_ref):
    dinv = _dinv_block(d0_ref, d1_ref)
    conv = dinv * (s_ref[0] + s_ref[1]) + b_ref[...]
    h = jnp.maximum(conv, 0.0)
    h2 = jnp.dot(h, w_ref[...], preferred_element_type=jnp.float32)
    out_ref[...] = h2 * dinv


def _final_body(s_ref, d0_ref, d1_ref, b_ref, out_ref):
    dinv = _dinv_block(d0_ref, d1_ref)
    out_ref[...] = dinv * (s_ref[0] + s_ref[1]) + b_ref[...]


RB = 1000
_GRID = N // RB


def _tc_call(body, in_specs, out_shape):
    return pl.pallas_call(
        body,
        grid=(_GRID,),
        in_specs=in_specs,
        out_specs=pl.BlockSpec((RB, D), lambda i: (i, 0)),
        out_shape=jax.ShapeDtypeStruct(out_shape, jnp.float32),
    )


_spec_rows = pl.BlockSpec((RB, D), lambda i: (i, 0))
_spec_w = pl.BlockSpec((D, D), lambda i: (0, 0))
_spec_degc = pl.BlockSpec((1, RB, 1), lambda i: (i, 0, 0))
_spec_b = pl.BlockSpec((1, D), lambda i: (0, 0))
_spec_s = pl.BlockSpec((NC, RB, D), lambda i: (0, i, 0))


def kernel(x, edge_index, emb_table, W1, b1, Wmu, bmu, Wlv, blv):
    f32 = jnp.float32
    src1 = edge_index[0]
    dst3 = edge_index[1].reshape(NW, CH, K)
    x5 = x.reshape(NS, XNJ, XCH)
    ones1 = jnp.ones((K,), f32)
    zero_d = jnp.zeros((NS, RPT, D), f32)
    W2 = jnp.concatenate([Wmu, Wlv], axis=1)
    b2 = jnp.concatenate([bmu, blv]).reshape(1, D)
    b1r = b1.reshape(1, D)

    deg1d, emb4 = _get_deg_kernel()(dst3, x5, emb_table, ones1)
    d0 = deg1d[:N].reshape(_GRID, RB, 1)
    d1 = deg1d[N:].reshape(_GRID, RB, 1)
    emb = emb4.reshape(N, D)

    h1p = _tc_call(
        _mm_scale_body, [_spec_rows, _spec_w, _spec_degc, _spec_degc], (N, D)
    )(emb, W1, d0, d1)

    s1 = _get_msgpass_kernel()(h1p, h1p.reshape(NS, RPT, D), src1, dst3, zero_d)
    s1 = s1.reshape(NC, N, D)

    h2p = _tc_call(
        _mid_body, [_spec_s, _spec_degc, _spec_degc, _spec_b, _spec_w], (N, D)
    )(s1, d0, d1, b1r, W2)

    s2 = _get_msgpass_kernel()(h2p, h2p.reshape(NS, RPT, D), src1, dst3, zero_d)
    s2 = s2.reshape(NC, N, D)

    out = _tc_call(
        _final_body, [_spec_s, _spec_degc, _spec_degc, _spec_b], (N, D)
    )(s2, d0, d1, b2)

    return (out[:, : D // 2], out[:, D // 2 :])


# speedup vs baseline: 1.2914x; 1.2914x over previous
import functools

import jax
import jax.numpy as jnp
from jax import lax
from jax.experimental import pallas as pl
from jax.experimental.pallas import tpu as pltpu
from jax.experimental.pallas import tpu_sc as plsc

N = 10000
E = 320000
D = 128
NC, NS = 2, 16
NW = NC * NS
EW = E // NW
K = 80
CH = EW // K
OUT_OFF = 624
OUT_WIN = 640
XCH = 128
XNJ = OUT_WIN // XCH


def _sc_mesh():
    return plsc.VectorSubcoreMesh(
        core_axis_name="c", subcore_axis_name="s", num_cores=NC, num_subcores=NS
    )


@functools.cache
def _get_deg_kernel():
    return functools.partial(
        pl.kernel,
        out_type=(
            jax.ShapeDtypeStruct((NC * N,), jnp.float32),
            jax.ShapeDtypeStruct((N, D), jnp.float32),
        ),
        mesh=_sc_mesh(),
        scratch_types=[
            pltpu.VMEM((CH, K), jnp.int32),
            pltpu.VMEM((K,), jnp.float32),
            pltpu.VMEM((OUT_WIN,), jnp.int32),
            pltpu.VMEM((XCH, D), jnp.float32),
            pltpu.VMEM((OUT_WIN,), jnp.float32),
            pltpu.VMEM_SHARED((N,), jnp.float32),
            pltpu.SemaphoreType.DMA,
        ],
    )(_deg_body)


def _deg_body(dst_hbm, x_hbm, emb_hbm, ones_hbm, deg_out, emb_out,
              dst_v, ones_v, x_v, erows_v, stage_v, acc_sh, sem):
    c = lax.axis_index("c")
    s = lax.axis_index("s")
    wid = c * NS + s
    pltpu.sync_copy(dst_hbm.at[wid], dst_v)
    pltpu.sync_copy(ones_hbm, ones_v)
    for j in range(OUT_WIN // 16):
        stage_v[pl.ds(16 * j, 16)] = jnp.zeros((16,), jnp.float32)
    pltpu.sync_copy(stage_v, acc_sh.at[pl.ds(s * OUT_OFF, OUT_WIN)])

    @pl.when(c == 0)
    def _():
        pltpu.sync_copy(x_hbm.at[pl.ds(s * OUT_OFF, OUT_WIN)], x_v)

    plsc.subcore_barrier()

    @pl.loop(0, CH)
    def _(i):
        pltpu.sync_copy(ones_v, acc_sh.at[dst_v.at[i]], add=True)

    @pl.when(c == 0)
    def _():
        for j in range(XNJ):
            pltpu.async_copy(
                emb_hbm.at[x_v.at[pl.ds(j * XCH, XCH)]], erows_v, sem).wait()
            pltpu.sync_copy(
                erows_v, emb_out.at[pl.ds(s * OUT_OFF + j * XCH, XCH)])

    plsc.subcore_barrier()
    pltpu.sync_copy(acc_sh.at[pl.ds(s * OUT_OFF, OUT_WIN)], stage_v)
    pltpu.sync_copy(stage_v, deg_out.at[pl.ds(c * N + s * OUT_OFF, OUT_WIN)])


@functools.cache
def _get_msgpass_kernel():
    return functools.partial(
        pl.kernel,
        out_type=jax.ShapeDtypeStruct((NC, N, D), jnp.float32),
        mesh=_sc_mesh(),
        scratch_types=[
            pltpu.VMEM((EW,), jnp.int32),
            pltpu.VMEM((CH, K), jnp.int32),
            pltpu.VMEM((K, D), jnp.float32),
            pltpu.VMEM((K, D), jnp.float32),
            pltpu.VMEM_SHARED((N, D), jnp.float32),
            pltpu.SemaphoreType.DMA,
            pltpu.SemaphoreType.DMA,
        ],
    )(_msgpass_body)


def _msgpass_body(h_hbm, src_hbm, dst_hbm, zero_hbm, out_hbm,
                  src_v, dst_v, rows_a, rows_b, acc_sh, sem_a, sem_b):
    c = lax.axis_index("c")
    s = lax.axis_index("s")
    wid = c * NS + s
    win = pl.ds(s * OUT_OFF, OUT_WIN)

    def _gather(i, buf, sem):
        pltpu.async_copy(h_hbm.at[src_v.at[pl.ds(i * K, K)]], buf, sem)

    def _gwait(i, buf, sem):
        pltpu.make_async_copy(
            h_hbm.at[src_v.at[pl.ds(i * K, K)]], buf, sem).wait()

    pltpu.sync_copy(src_hbm.at[pl.ds(wid * EW, EW)], src_v)
    _gather(0, rows_a, sem_a)
    _gather(1, rows_b, sem_b)
    pltpu.sync_copy(dst_hbm.at[wid], dst_v)
    @pl.when(c == 0)
    def _():
        pltpu.sync_copy(h_hbm.at[win], acc_sh.at[win])

    @pl.when(c != 0)
    def _():
        pltpu.sync_copy(zero_hbm.at[win], acc_sh.at[win])

    plsc.subcore_barrier()

    @pl.loop(0, CH // 2)
    def _(t):
        i0 = 2 * t
        _gwait(i0, rows_a, sem_a)
        pltpu.sync_copy(rows_a, acc_sh.at[dst_v.at[i0]], add=True)
        _gather(i0 + 2, rows_a, sem_a)
        _gwait(i0 + 1, rows_b, sem_b)
        pltpu.sync_copy(rows_b, acc_sh.at[dst_v.at[i0 + 1]], add=True)

        @pl.when(t < CH // 2 - 1)
        def _():
            _gather(i0 + 3, rows_b, sem_b)

    _gwait(CH - 1, rows_a, sem_a)
    pltpu.sync_copy(rows_a, acc_sh.at[dst_v.at[CH - 1]], add=True)

    plsc.subcore_barrier()
    pltpu.sync_copy(acc_sh.at[win], out_hbm.at[c, win])


RB = 1000
_GRID = N // RB


def _dinv_block(d0_ref, d1_ref):
    d = 1.0 + d0_ref[0] + d1_ref[0]
    return lax.rsqrt(d)


def _mm_scale_body(emb_ref, w_ref, d0_ref, d1_ref, out_ref):
    dinv = _dinv_block(d0_ref, d1_ref)
    h = jnp.dot(emb_ref[...], w_ref[...], preferred_element_type=jnp.float32)
    out_ref[...] = h * dinv


def _mid_body(s_ref, d0_ref, d1_ref, b_ref, w_ref, out_ref):
    dinv = _dinv_block(d0_ref, d1_ref)
    conv = dinv * (s_ref[0] + s_ref[1]) + b_ref[...]
    h = jnp.maximum(conv, 0.0)
    h2 = jnp.dot(h, w_ref[...], preferred_element_type=jnp.float32)
    out_ref[...] = h2 * dinv


def _final_body(s_ref, d0_ref, d1_ref, b_ref, mu_ref, lv_ref):
    dinv = _dinv_block(d0_ref, d1_ref)
    out = dinv * (s_ref[0] + s_ref[1]) + b_ref[...]
    mu_ref[...] = out[:, : D // 2]
    lv_ref[...] = out[:, D // 2 :]


def _tc_call(body, in_specs, out_specs, out_shape):
    return pl.pallas_call(
        body,
        grid=(_GRID,),
        in_specs=in_specs,
        out_specs=out_specs,
        out_shape=out_shape,
    )


_spec_rows = pl.BlockSpec((RB, D), lambda i: (i, 0))
_spec_w = pl.BlockSpec((D, D), lambda i: (0, 0))
_spec_degm = pl.BlockSpec((1, RB, 1), lambda i: (i, 0, 0))
_spec_b = pl.BlockSpec((1, D), lambda i: (0, 0))
_spec_s = pl.BlockSpec((NC, RB, D), lambda i: (0, i, 0))
_spec_half = pl.BlockSpec((RB, D // 2), lambda i: (i, 0))


def kernel(x, edge_index, emb_table, W1, b1, Wmu, bmu, Wlv, blv):
    f32 = jnp.float32
    src1 = edge_index[0]
    dst3 = edge_index[1].reshape(NW, CH, K)
    ones1 = jnp.ones((K,), f32)
    zero_d = jnp.zeros((N, D), f32)
    W2 = jnp.concatenate([Wmu, Wlv], axis=1)
    b2 = jnp.concatenate([bmu, blv]).reshape(1, D)
    b1r = b1.reshape(1, D)

    deg1d, emb = _get_deg_kernel()(dst3, x, emb_table, ones1)
    d0 = deg1d[:N].reshape(_GRID, RB, 1)
    d1 = deg1d[N:].reshape(_GRID, RB, 1)

    h1p = _tc_call(
        _mm_scale_body,
        [_spec_rows, _spec_w, _spec_degm, _spec_degm],
        pl.BlockSpec((RB, D), lambda i: (i, 0)),
        jax.ShapeDtypeStruct((N, D), f32),
    )(emb, W1, d0, d1)

    s1 = _get_msgpass_kernel()(h1p, src1, dst3, zero_d)

    h2p = _tc_call(
        _mid_body,
        [_spec_s, _spec_degm, _spec_degm, _spec_b, _spec_w],
        pl.BlockSpec((RB, D), lambda i: (i, 0)),
        jax.ShapeDtypeStruct((N, D), f32),
    )(s1, d0, d1, b1r, W2)

    s2 = _get_msgpass_kernel()(h2p, src1, dst3, zero_d)

    mu, lv = _tc_call(
        _final_body,
        [_spec_s, _spec_degm, _spec_degm, _spec_b],
        [_spec_half, _spec_half],
        (
            jax.ShapeDtypeStruct((N, D // 2), f32),
            jax.ShapeDtypeStruct((N, D // 2), f32),
        ),
    )(s2, d0, d1, b2)

    return (mu, lv)


# speedup vs baseline: 1.3906x; 1.0769x over previous
import functools

import jax
import jax.numpy as jnp
from jax import lax
from jax.experimental import pallas as pl
from jax.experimental.pallas import tpu as pltpu
from jax.experimental.pallas import tpu_sc as plsc

N = 10000
E = 320000
D = 128
NC, NS = 2, 16
NW = NC * NS
EW = E // NW
K = 128
CH = -(-EW // K)
EWP = CH * K
NACC = N + 8
SHALF = (CH // 2 + 1) * K
CH_A = SHALF // K
CH_B = CH - CH_A
OUT_OFF = 624
OUT_WIN = 640
XCH = 128
XNJ = OUT_WIN // XCH


def _sc_mesh():
    return plsc.VectorSubcoreMesh(
        core_axis_name="c", subcore_axis_name="s", num_cores=NC, num_subcores=NS
    )


@functools.cache
def _get_deg_kernel():
    return functools.partial(
        pl.kernel,
        out_type=(
            jax.ShapeDtypeStruct((NC * N,), jnp.float32),
            jax.ShapeDtypeStruct((N, D), jnp.float32),
        ),
        mesh=_sc_mesh(),
        scratch_types=[
            pltpu.VMEM((CH, K), jnp.int32),
            pltpu.VMEM((K,), jnp.float32),
            pltpu.VMEM((OUT_WIN,), jnp.int32),
            pltpu.VMEM((XCH, D), jnp.float32),
            pltpu.VMEM((XCH, D), jnp.float32),
            pltpu.VMEM((OUT_WIN,), jnp.float32),
            pltpu.VMEM_SHARED((NACC,), jnp.float32),
            pltpu.SemaphoreType.DMA,
        ],
    )(_deg_body)


def _deg_body(dst_hbm, x_hbm, emb_hbm, ones_hbm, deg_out, emb_out,
              dst_v, ones_v, x_v, erows_a, erows_b, stage_v, acc_sh, sem):
    c = lax.axis_index("c")
    s = lax.axis_index("s")
    wid = c * NS + s

    def _egather(j, buf):
        idx = x_v.at[pl.ds(j * XCH, XCH)]
        pltpu.async_copy(emb_hbm.at[idx], buf, sem)

    def _ewait(j, buf):
        idx = x_v.at[pl.ds(j * XCH, XCH)]
        pltpu.make_async_copy(emb_hbm.at[idx], buf, sem).wait()

    pltpu.sync_copy(dst_hbm.at[wid], dst_v)
    pltpu.sync_copy(ones_hbm, ones_v)

    @pl.when(c == 0)
    def _():
        pltpu.sync_copy(x_hbm.at[pl.ds(s * OUT_OFF, OUT_WIN)], x_v)
        _egather(0, erows_a)

    for j in range(OUT_WIN // 16):
        stage_v[pl.ds(16 * j, 16)] = jnp.zeros((16,), jnp.float32)
    pltpu.sync_copy(stage_v, acc_sh.at[pl.ds(s * OUT_OFF, OUT_WIN)])

    plsc.subcore_barrier()

    @pl.loop(0, CH)
    def _(i):
        pltpu.sync_copy(ones_v, acc_sh.at[dst_v.at[i]], add=True)

    @pl.when(c == 0)
    def _():
        bufs = (erows_a, erows_b)
        for j in range(XNJ):
            _ewait(j, bufs[j % 2])
            if j + 1 < XNJ:
                _egather(j + 1, bufs[(j + 1) % 2])
            pltpu.sync_copy(
                bufs[j % 2], emb_out.at[pl.ds(s * OUT_OFF + j * XCH, XCH)])

    plsc.subcore_barrier()
    pltpu.sync_copy(acc_sh.at[pl.ds(s * OUT_OFF, OUT_WIN)], stage_v)
    pltpu.sync_copy(stage_v, deg_out.at[pl.ds(c * N + s * OUT_OFF, OUT_WIN)])


@functools.cache
def _get_msgpass_kernel():
    return functools.partial(
        pl.kernel,
        out_type=jax.ShapeDtypeStruct((NC, N, D), jnp.float32),
        mesh=_sc_mesh(),
        scratch_types=[
            pltpu.VMEM((SHALF,), jnp.int32),
            pltpu.VMEM((CH, K), jnp.int32),
            pltpu.VMEM((K, D), jnp.float32),
            pltpu.VMEM((K, D), jnp.float32),
            pltpu.VMEM_SHARED((NACC, D), jnp.float32),
            pltpu.SemaphoreType.DMA,
            pltpu.SemaphoreType.DMA,
        ],
    )(_msgpass_body)


def _msgpass_body(h_hbm, src_hbm, dst_hbm, zero_hbm, out_hbm,
                  src_v, dst_v, rows_a, rows_b, acc_sh, sem_a, sem_b):
    c = lax.axis_index("c")
    s = lax.axis_index("s")
    wid = c * NS + s
    win = pl.ds(s * OUT_OFF, OUT_WIN)

    def _gather(i, base, buf, sem):
        idx = src_v.at[pl.ds((i - base) * K, K)]
        pltpu.async_copy(h_hbm.at[idx], buf, sem)

    def _gwait(i, base, buf, sem):
        idx = src_v.at[pl.ds((i - base) * K, K)]
        pltpu.make_async_copy(h_hbm.at[idx], buf, sem).wait()

    pltpu.sync_copy(src_hbm.at[pl.ds(wid * EWP, SHALF)], src_v)
    _gather(0, 0, rows_a, sem_a)
    _gather(1, 0, rows_b, sem_b)
    pltpu.sync_copy(dst_hbm.at[wid], dst_v)
    @pl.when(c == 0)
    def _():
        pltpu.sync_copy(h_hbm.at[win], acc_sh.at[win])

    @pl.when(c != 0)
    def _():
        pltpu.sync_copy(zero_hbm.at[win], acc_sh.at[win])

    plsc.subcore_barrier()

    @pl.loop(0, CH_A // 2 - 1)
    def _(t):
        i0 = 2 * t
        _gwait(i0, 0, rows_a, sem_a)
        pltpu.sync_copy(rows_a, acc_sh.at[dst_v.at[i0]], add=True)
        _gather(i0 + 2, 0, rows_a, sem_a)
        _gwait(i0 + 1, 0, rows_b, sem_b)
        pltpu.sync_copy(rows_b, acc_sh.at[dst_v.at[i0 + 1]], add=True)

        @pl.when(t < CH_A // 2 - 1)
        def _():
            _gather(i0 + 3, 0, rows_b, sem_b)

    _gwait(CH_A - 2, 0, rows_a, sem_a)
    pltpu.sync_copy(rows_a, acc_sh.at[dst_v.at[CH_A - 2]], add=True)
    _gwait(CH_A - 1, 0, rows_b, sem_b)
    pltpu.sync_copy(rows_b, acc_sh.at[dst_v.at[CH_A - 1]], add=True)

    pltpu.sync_copy(
        src_hbm.at[pl.ds(wid * EWP + SHALF, EWP - SHALF)],
        src_v.at[pl.ds(0, EWP - SHALF)])
    _gather(CH_A, CH_A, rows_a, sem_a)
    _gather(CH_A + 1, CH_A, rows_b, sem_b)

    @pl.loop(0, CH_B // 2)
    def _(t):
        i0 = CH_A + 2 * t
        _gwait(i0, CH_A, rows_a, sem_a)
        pltpu.sync_copy(rows_a, acc_sh.at[dst_v.at[i0]], add=True)
        _gather(i0 + 2, CH_A, rows_a, sem_a)
        _gwait(i0 + 1, CH_A, rows_b, sem_b)
        pltpu.sync_copy(rows_b, acc_sh.at[dst_v.at[i0 + 1]], add=True)

        @pl.when(t < CH_B // 2 - 1)
        def _():
            _gather(i0 + 3, CH_A, rows_b, sem_b)

    _gwait(CH - 1, CH_A, rows_a, sem_a)
    pltpu.sync_copy(rows_a, acc_sh.at[dst_v.at[CH - 1]], add=True)

    plsc.subcore_barrier()
    pltpu.sync_copy(acc_sh.at[win], out_hbm.at[c, win])


RB = 1000
_GRID = N // RB


def _dinv_block(d0_ref, d1_ref):
    d = 1.0 + d0_ref[0] + d1_ref[0]
    return lax.rsqrt(d)


def _mm_scale_body(emb_ref, w_ref, d0_ref, d1_ref, out_ref):
    dinv = _dinv_block(d0_ref, d1_ref)
    h = jnp.dot(emb_ref[...], w_ref[...], preferred_element_type=jnp.float32)
    out_ref[...] = h * dinv


def _mid_body(s_ref, d0_ref, d1_ref, b_ref, w_ref, out_ref):
    dinv = _dinv_block(d0_ref, d1_ref)
    conv = dinv * (s_ref[0] + s_ref[1]) + b_ref[...]
    h = jnp.maximum(conv, 0.0)
    h2 = jnp.dot(h, w_ref[...], preferred_element_type=jnp.float32)
    out_ref[...] = h2 * dinv


def _final_body(s_ref, d0_ref, d1_ref, b_ref, mu_ref, lv_ref):
    dinv = _dinv_block(d0_ref, d1_ref)
    out = dinv * (s_ref[0] + s_ref[1]) + b_ref[...]
    mu_ref[...] = out[:, : D // 2]
    lv_ref[...] = out[:, D // 2 :]


def _tc_call(body, in_specs, out_specs, out_shape):
    return pl.pallas_call(
        body,
        grid=(_GRID,),
        in_specs=in_specs,
        out_specs=out_specs,
        out_shape=out_shape,
    )


_spec_rows = pl.BlockSpec((RB, D), lambda i: (i, 0))
_spec_w = pl.BlockSpec((D, D), lambda i: (0, 0))
_spec_degm = pl.BlockSpec((1, RB, 1), lambda i: (i, 0, 0))
_spec_b = pl.BlockSpec((1, D), lambda i: (0, 0))
_spec_s = pl.BlockSpec((NC, RB, D), lambda i: (0, i, 0))
_spec_half = pl.BlockSpec((RB, D // 2), lambda i: (i, 0))


def kernel(x, edge_index, emb_table, W1, b1, Wmu, bmu, Wlv, blv):
    f32 = jnp.float32
    i32 = jnp.int32
    npad = EWP - EW
    pad_src = (jnp.arange(NW, dtype=i32)[:, None] * 113
               + jnp.arange(npad, dtype=i32)[None, :]) % N
    src1 = jnp.concatenate(
        [edge_index[0].reshape(NW, EW), pad_src], axis=1).reshape(-1)
    dst3 = jnp.concatenate(
        [edge_index[1].reshape(NW, EW),
         jnp.full((NW, npad), N, dtype=i32)], axis=1).reshape(NW, CH, K)
    ones1 = jnp.ones((K,), f32)
    zero_d = jnp.zeros((N, D), f32)
    W2 = jnp.concatenate([Wmu, Wlv], axis=1)
    b2 = jnp.concatenate([bmu, blv]).reshape(1, D)
    b1r = b1.reshape(1, D)

    deg1d, emb = _get_deg_kernel()(dst3, x, emb_table, ones1)
    d0 = deg1d[:N].reshape(_GRID, RB, 1)
    d1 = deg1d[N:].reshape(_GRID, RB, 1)

    h1p = _tc_call(
        _mm_scale_body,
        [_spec_rows, _spec_w, _spec_degm, _spec_degm],
        pl.BlockSpec((RB, D), lambda i: (i, 0)),
        jax.ShapeDtypeStruct((N, D), f32),
    )(emb, W1, d0, d1)

    s1 = _get_msgpass_kernel()(h1p, src1, dst3, zero_d)

    h2p = _tc_call(
        _mid_body,
        [_spec_s, _spec_degm, _spec_degm, _spec_b, _spec_w],
        pl.BlockSpec((RB, D), lambda i: (i, 0)),
        jax.ShapeDtypeStruct((N, D), f32),
    )(s1, d0, d1, b1r, W2)

    s2 = _get_msgpass_kernel()(h2p, src1, dst3, zero_d)

    mu, lv = _tc_call(
        _final_body,
        [_spec_s, _spec_degm, _spec_degm, _spec_b],
        [_spec_half, _spec_half],
        (
            jax.ShapeDtypeStruct((N, D // 2), f32),
            jax.ShapeDtypeStruct((N, D // 2), f32),
        ),
    )(s2, d0, d1, b2)

    return (mu, lv)


# speedup vs baseline: 1.3921x; 1.0010x over previous
import functools

import jax
import jax.numpy as jnp
from jax import lax
from jax.experimental import pallas as pl
from jax.experimental.pallas import tpu as pltpu
from jax.experimental.pallas import tpu_sc as plsc

N = 10000
E = 320000
D = 128
NC, NS = 2, 16
NW = NC * NS
EW = E // NW
K = 128
CH = -(-EW // K)
EWP = CH * K
NACC = N + 128
SHALF = (CH // 2 + 1) * K
CH_A = SHALF // K
CH_B = CH - CH_A
OUT_OFF = 624
OUT_WIN = 640
XCH = 128
XNJ = OUT_WIN // XCH


def _sc_mesh():
    return plsc.VectorSubcoreMesh(
        core_axis_name="c", subcore_axis_name="s", num_cores=NC, num_subcores=NS
    )


@functools.cache
def _get_deg_kernel():
    return functools.partial(
        pl.kernel,
        out_type=(
            jax.ShapeDtypeStruct((NC * N,), jnp.float32),
            jax.ShapeDtypeStruct((N, D), jnp.float32),
        ),
        mesh=_sc_mesh(),
        scratch_types=[
            pltpu.VMEM((CH, K), jnp.int32),
            pltpu.VMEM((K,), jnp.float32),
            pltpu.VMEM((OUT_WIN,), jnp.int32),
            pltpu.VMEM((XCH, D), jnp.float32),
            pltpu.VMEM((XCH, D), jnp.float32),
            pltpu.VMEM((OUT_WIN,), jnp.float32),
            pltpu.VMEM_SHARED((NACC,), jnp.float32),
            pltpu.SemaphoreType.DMA,
        ],
    )(_deg_body)


def _deg_body(dst_hbm, x_hbm, emb_hbm, ones_hbm, deg_out, emb_out,
              dst_v, ones_v, x_v, erows_a, erows_b, stage_v, acc_sh, sem):
    c = lax.axis_index("c")
    s = lax.axis_index("s")
    wid = c * NS + s

    def _egather(j, buf):
        idx = x_v.at[pl.ds(j * XCH, XCH)]
        pltpu.async_copy(emb_hbm.at[idx], buf, sem)

    def _ewait(j, buf):
        idx = x_v.at[pl.ds(j * XCH, XCH)]
        pltpu.make_async_copy(emb_hbm.at[idx], buf, sem).wait()

    pltpu.sync_copy(dst_hbm.at[wid], dst_v)
    pltpu.sync_copy(ones_hbm, ones_v)

    @pl.when(c == 0)
    def _():
        pltpu.sync_copy(x_hbm.at[pl.ds(s * OUT_OFF, OUT_WIN)], x_v)
        _egather(0, erows_a)

    for j in range(OUT_WIN // 16):
        stage_v[pl.ds(16 * j, 16)] = jnp.zeros((16,), jnp.float32)
    pltpu.sync_copy(stage_v, acc_sh.at[pl.ds(s * OUT_OFF, OUT_WIN)])

    plsc.subcore_barrier()

    @pl.loop(0, CH)
    def _(i):
        pltpu.sync_copy(ones_v, acc_sh.at[dst_v.at[i]], add=True)

    @pl.when(c == 0)
    def _():
        bufs = (erows_a, erows_b)
        for j in range(XNJ):
            _ewait(j, bufs[j % 2])
            if j + 1 < XNJ:
                _egather(j + 1, bufs[(j + 1) % 2])
            pltpu.sync_copy(
                bufs[j % 2], emb_out.at[pl.ds(s * OUT_OFF + j * XCH, XCH)])

    plsc.subcore_barrier()
    pltpu.sync_copy(acc_sh.at[pl.ds(s * OUT_OFF, OUT_WIN)], stage_v)
    pltpu.sync_copy(stage_v, deg_out.at[pl.ds(c * N + s * OUT_OFF, OUT_WIN)])


@functools.cache
def _get_msgpass_kernel():
    return functools.partial(
        pl.kernel,
        out_type=jax.ShapeDtypeStruct((NC, N, D), jnp.float32),
        mesh=_sc_mesh(),
        scratch_types=[
            pltpu.VMEM((SHALF,), jnp.int32),
            pltpu.VMEM((CH, K), jnp.int32),
            pltpu.VMEM((K, D), jnp.float32),
            pltpu.VMEM((K, D), jnp.float32),
            pltpu.VMEM_SHARED((NACC, D), jnp.float32),
            pltpu.SemaphoreType.DMA,
            pltpu.SemaphoreType.DMA,
        ],
    )(_msgpass_body)


def _msgpass_body(h_hbm, src_hbm, dst_hbm, zero_hbm, out_hbm,
                  src_v, dst_v, rows_a, rows_b, acc_sh, sem_a, sem_b):
    c = lax.axis_index("c")
    s = lax.axis_index("s")
    wid = c * NS + s
    win = pl.ds(s * OUT_OFF, OUT_WIN)

    def _gather(i, base, buf, sem):
        idx = src_v.at[pl.ds((i - base) * K, K)]
        pltpu.async_copy(h_hbm.at[idx], buf, sem)

    def _gwait(i, base, buf, sem):
        idx = src_v.at[pl.ds((i - base) * K, K)]
        pltpu.make_async_copy(h_hbm.at[idx], buf, sem).wait()

    pltpu.sync_copy(src_hbm.at[pl.ds(wid * EWP, SHALF)], src_v)
    _gather(0, 0, rows_a, sem_a)
    _gather(1, 0, rows_b, sem_b)
    pltpu.sync_copy(dst_hbm.at[wid], dst_v)
    @pl.when(c == 0)
    def _():
        pltpu.sync_copy(h_hbm.at[win], acc_sh.at[win])

    @pl.when(c != 0)
    def _():
        pltpu.sync_copy(zero_hbm.at[win], acc_sh.at[win])

    plsc.subcore_barrier()

    @pl.loop(0, CH_A // 2 - 1)
    def _(t):
        i0 = 2 * t
        _gwait(i0, 0, rows_a, sem_a)
        pltpu.sync_copy(rows_a, acc_sh.at[dst_v.at[i0]], add=True)
        _gather(i0 + 2, 0, rows_a, sem_a)
        _gwait(i0 + 1, 0, rows_b, sem_b)
        pltpu.sync_copy(rows_b, acc_sh.at[dst_v.at[i0 + 1]], add=True)

        @pl.when(t < CH_A // 2 - 1)
        def _():
            _gather(i0 + 3, 0, rows_b, sem_b)

    _gwait(CH_A - 2, 0, rows_a, sem_a)
    pltpu.sync_copy(rows_a, acc_sh.at[dst_v.at[CH_A - 2]], add=True)
    _gwait(CH_A - 1, 0, rows_b, sem_b)
    pltpu.sync_copy(rows_b, acc_sh.at[dst_v.at[CH_A - 1]], add=True)

    pltpu.sync_copy(
        src_hbm.at[pl.ds(wid * EWP + SHALF, EWP - SHALF)],
        src_v.at[pl.ds(0, EWP - SHALF)])
    _gather(CH_A, CH_A, rows_a, sem_a)
    _gather(CH_A + 1, CH_A, rows_b, sem_b)

    @pl.loop(0, CH_B // 2)
    def _(t):
        i0 = CH_A + 2 * t
        _gwait(i0, CH_A, rows_a, sem_a)
        pltpu.sync_copy(rows_a, acc_sh.at[dst_v.at[i0]], add=True)
        _gather(i0 + 2, CH_A, rows_a, sem_a)
        _gwait(i0 + 1, CH_A, rows_b, sem_b)
        pltpu.sync_copy(rows_b, acc_sh.at[dst_v.at[i0 + 1]], add=True)

        @pl.when(t < CH_B // 2 - 1)
        def _():
            _gather(i0 + 3, CH_A, rows_b, sem_b)

    _gwait(CH - 1, CH_A, rows_a, sem_a)
    pltpu.sync_copy(rows_a, acc_sh.at[dst_v.at[CH - 1]], add=True)

    plsc.subcore_barrier()
    pltpu.sync_copy(acc_sh.at[win], out_hbm.at[c, win])


RB = 1000
_GRID = N // RB


def _dinv_block(d0_ref, d1_ref):
    d = 1.0 + d0_ref[0] + d1_ref[0]
    return lax.rsqrt(d)


def _mm_scale_body(emb_ref, w_ref, d0_ref, d1_ref, out_ref):
    dinv = _dinv_block(d0_ref, d1_ref)
    h = jnp.dot(emb_ref[...], w_ref[...], preferred_element_type=jnp.float32)
    out_ref[...] = h * dinv


def _mid_body(s_ref, d0_ref, d1_ref, b_ref, w_ref, out_ref):
    dinv = _dinv_block(d0_ref, d1_ref)
    conv = dinv * (s_ref[0] + s_ref[1]) + b_ref[...]
    h = jnp.maximum(conv, 0.0)
    h2 = jnp.dot(h, w_ref[...], preferred_element_type=jnp.float32)
    out_ref[...] = h2 * dinv


def _final_body(s_ref, d0_ref, d1_ref, b_ref, mu_ref, lv_ref):
    dinv = _dinv_block(d0_ref, d1_ref)
    out = dinv * (s_ref[0] + s_ref[1]) + b_ref[...]
    mu_ref[...] = out[:, : D // 2]
    lv_ref[...] = out[:, D // 2 :]


def _tc_call(body, in_specs, out_specs, out_shape):
    return pl.pallas_call(
        body,
        grid=(_GRID,),
        in_specs=in_specs,
        out_specs=out_specs,
        out_shape=out_shape,
    )


_spec_rows = pl.BlockSpec((RB, D), lambda i: (i, 0))
_spec_w = pl.BlockSpec((D, D), lambda i: (0, 0))
_spec_degm = pl.BlockSpec((1, RB, 1), lambda i: (i, 0, 0))
_spec_b = pl.BlockSpec((1, D), lambda i: (0, 0))
_spec_s = pl.BlockSpec((NC, RB, D), lambda i: (0, i, 0))
_spec_half = pl.BlockSpec((RB, D // 2), lambda i: (i, 0))


def kernel(x, edge_index, emb_table, W1, b1, Wmu, bmu, Wlv, blv):
    f32 = jnp.float32
    i32 = jnp.int32
    npad = EWP - EW
    pad_src = (jnp.arange(NW, dtype=i32)[:, None] * 113
               + jnp.arange(npad, dtype=i32)[None, :]) % N
    src1 = jnp.concatenate(
        [edge_index[0].reshape(NW, EW), pad_src], axis=1).reshape(-1)
    pad_dst = jnp.broadcast_to(
        N + jnp.arange(npad, dtype=i32)[None, :], (NW, npad))
    dst3 = jnp.concatenate(
        [edge_index[1].reshape(NW, EW), pad_dst], axis=1).reshape(NW, CH, K)
    ones1 = jnp.ones((K,), f32)
    zero_d = jnp.zeros((N, D), f32)
    W2 = jnp.concatenate([Wmu, Wlv], axis=1)
    b2 = jnp.concatenate([bmu, blv]).reshape(1, D)
    b1r = b1.reshape(1, D)

    deg1d, emb = _get_deg_kernel()(dst3, x, emb_table, ones1)
    d0 = deg1d[:N].reshape(_GRID, RB, 1)
    d1 = deg1d[N:].reshape(_GRID, RB, 1)

    h1p = _tc_call(
        _mm_scale_body,
        [_spec_rows, _spec_w, _spec_degm, _spec_degm],
        pl.BlockSpec((RB, D), lambda i: (i, 0)),
        jax.ShapeDtypeStruct((N, D), f32),
    )(emb, W1, d0, d1)

    s1 = _get_msgpass_kernel()(h1p, src1, dst3, zero_d)

    h2p = _tc_call(
        _mid_body,
        [_spec_s, _spec_degm, _spec_degm, _spec_b, _spec_w],
        pl.BlockSpec((RB, D), lambda i: (i, 0)),
        jax.ShapeDtypeStruct((N, D), f32),
    )(s1, d0, d1, b1r, W2)

    s2 = _get_msgpass_kernel()(h2p, src1, dst3, zero_d)

    mu, lv = _tc_call(
        _final_body,
        [_spec_s, _spec_degm, _spec_degm, _spec_b],
        [_spec_half, _spec_half],
        (
            jax.ShapeDtypeStruct((N, D // 2), f32),
            jax.ShapeDtypeStruct((N, D // 2), f32),
        ),
    )(s2, d0, d1, b2)

    return (mu, lv)
